# baseline (speedup 1.0000x reference)
import functools

import jax
import jax.numpy as jnp
from jax import lax
from jax.experimental import pallas as pl
from jax.experimental.pallas import tpu as pltpu
from jax.experimental.pallas import tpu_sc as plsc

N = 50000
E = 800000
C = 5
D = 20
F_PL = 16

BN = 2000
PCOLS = 128
ACOLS = 16
GCOLS = 128
N_PAD = 50688


def _prep_body(x_ref, wi_ref, wj_ref, be_ref, p_ref, a_ref):
    xb = x_ref[...]
    wi = wi_ref[...]
    wj = wj_ref[...]
    be = be_ref[...]
    a = jnp.sum(xb * wi[None], axis=-1) + be[None]
    b = jnp.sum(xb * wj[None], axis=-1)
    rows = xb.shape[0]
    parts = [xb[:, c, :] for c in range(C)]
    parts.extend([a, b, jnp.zeros((rows, PCOLS - 100 - 2 * C), jnp.float32)])
    p_ref[...] = jnp.concatenate(parts, axis=1)
    a_ref[...] = jnp.concatenate(
        [a, jnp.zeros((rows, ACOLS - C), jnp.float32)], axis=1)


def _prep(x, W_e, b_e):
    wi = W_e[:, 0, :D]
    wj = W_e[:, 0, D:]
    be = b_e[:, 0]
    return pl.pallas_call(
        _prep_body,
        grid=(N // BN,),
        in_specs=[
            pl.BlockSpec((BN, C, D), lambda i: (i, 0, 0)),
            pl.BlockSpec((C, D), lambda i: (0, 0)),
            pl.BlockSpec((C, D), lambda i: (0, 0)),
            pl.BlockSpec((C,), lambda i: (0,)),
        ],
        out_specs=[
            pl.BlockSpec((BN, PCOLS), lambda i: (i, 0)),
            pl.BlockSpec((BN, ACOLS), lambda i: (i, 0)),
        ],
        out_shape=[
            jax.ShapeDtypeStruct((N, PCOLS), jnp.float32),
            jax.ShapeDtypeStruct((N_PAD, ACOLS), jnp.float32),
        ],
    )(x, wi, wj, be)


def _mlp_body(x_ref, g_ref, w1x_ref, w1a_ref, b1_ref, w2_ref, b2_ref, o_ref):
    xb = x_ref[...]
    gb = g_ref[...]
    for c in range(C):
        ac = gb[:, c * D:(c + 1) * D]
        h1 = jnp.tanh(
            jax.lax.dot_general(xb[:, c, :], w1x_ref[c],
                                (((1,), (1,)), ((), ())),
                                preferred_element_type=jnp.float32)
            + jax.lax.dot_general(ac, w1a_ref[c],
                                  (((1,), (1,)), ((), ())),
                                  preferred_element_type=jnp.float32)
            + b1_ref[c][None])
        h2 = jnp.tanh(
            jax.lax.dot_general(h1, w2_ref[c],
                                (((1,), (1,)), ((), ())),
                                preferred_element_type=jnp.float32)
            + b2_ref[c][None])
        o_ref[:, c, :] = h2


def _mlp(x, aggr, W1, b1, W2, b2):
    w1x = W1[:, :, :D]
    w1a = W1[:, :, D:]
    return pl.pallas_call(
        _mlp_body,
        grid=(N // BN,),
        in_specs=[
            pl.BlockSpec((BN, C, D), lambda i: (i, 0, 0)),
            pl.BlockSpec((BN, GCOLS), lambda i: (i, 0)),
            pl.BlockSpec((C, F_PL, D), lambda i: (0, 0, 0)),
            pl.BlockSpec((C, F_PL, D), lambda i: (0, 0, 0)),
            pl.BlockSpec((C, F_PL), lambda i: (0, 0)),
            pl.BlockSpec((C, F_PL, F_PL), lambda i: (0, 0, 0)),
            pl.BlockSpec((C, F_PL), lambda i: (0, 0)),
        ],
        out_specs=pl.BlockSpec((BN, C, F_PL), lambda i: (i, 0, 0)),
        out_shape=jax.ShapeDtypeStruct((N, C, F_PL), jnp.float32),
    )(x, aggr, w1x, w1a, b1, W2, b2)


WROWS = 8448
NPASS = 3
NSC = 2
BSZ = 96
CH = 2000
E_TILE = E // 16
NCH = E_TILE // CH
NGRP = CH // 16
TROWS = WROWS // 16


def _edge_sc(P, A, src, dst):
    mesh = plsc.VectorSubcoreMesh(core_axis_name="c", subcore_axis_name="s")

    @functools.partial(
        pl.kernel, mesh=mesh,
        out_type=jax.ShapeDtypeStruct((N_PAD, GCOLS), jnp.float32),
        scratch_types=[
            pltpu.VMEM((CH,), jnp.int32),
            pltpu.VMEM((CH,), jnp.int32),
            pltpu.VMEM((BSZ + 64,), jnp.int32),
            pltpu.VMEM((BSZ,), jnp.int32),
            pltpu.VMEM((BSZ,), jnp.int32),
            pltpu.VMEM((BSZ,), jnp.int32),
            pltpu.VMEM((BSZ, PCOLS), jnp.float32),
            pltpu.VMEM((BSZ, PCOLS), jnp.float32),
            pltpu.VMEM((BSZ, GCOLS), jnp.float32),
            pltpu.VMEM((BSZ, GCOLS), jnp.float32),
            pltpu.VMEM_SHARED((WROWS + 16, GCOLS), jnp.float32),
            pltpu.SemaphoreType.DMA,
            pltpu.SemaphoreType.DMA,
        ],
        compiler_params=pltpu.CompilerParams(needs_layout_passes=False),
    )
    def k(p_hbm, a_hbm, src_hbm, dst_hbm, out_hbm,
          esrc, edst, pbuf, sidx, didx, dloc, prow, drow, msg, zbuf,
          win, sem, sems):
        cid = lax.axis_index("c")
        sid = lax.axis_index("s")
        iota = lax.iota(jnp.int32, 16)
        zi = jnp.zeros((16,), jnp.int32)
        zf = jnp.zeros((16,), jnp.float32)

        for g in range((BSZ + 64) // 16):
            pbuf[pl.ds(g * 16, 16)] = zi

        def _zrow(r, _):
            for kk in range(GCOLS // 16):
                zbuf[r, pl.ds(kk * 16, 16)] = zf
            return 0
        lax.fori_loop(0, BSZ, _zrow, 0)

        def flush(nact, lo):
            pltpu.make_async_copy(msg, win.at[dloc], sems).wait()
            for g in range(BSZ // 16):
                pk = pbuf[pl.ds(g * 16, 16)]
                s = pk & 0xFFFF
                d = (pk >> 16) & 0xFFFF
                act = (iota + (g * 16)) < nact
                sidx[pl.ds(g * 16, 16)] = s
                didx[pl.ds(g * 16, 16)] = d
                dloc[pl.ds(g * 16, 16)] = jnp.where(act, d - lo, WROWS + sid)
            d1 = pltpu.async_copy(p_hbm.at[didx], drow, sem)
            d2 = pltpu.async_copy(p_hbm.at[sidx], prow, sem)
            d1.wait()
            d2.wait()
            for g in range(BSZ // 16):
                rid = iota + (g * 16)
                ev = [plsc.load_gather(drow, [rid, zi + (100 + c)])
                      + plsc.load_gather(prow, [rid, zi + (105 + c)])
                      for c in range(C)]
                m = ev[0]
                for c in range(1, C):
                    m = jnp.maximum(m, ev[c])
                ex = [jnp.exp(e - m) for e in ev]
                s = ex[0]
                for c in range(1, C):
                    s = s + ex[c]
                w = [e / s for e in ex]

                @plsc.parallel_loop(0, D, unroll=10)
                def _(f):
                    for c in range(C):
                        colv = zi + (c * D) + f
                        xv = plsc.load_gather(prow, [rid, colv])
                        plsc.store_scatter(msg, [rid, colv], xv * w[c])
            pltpu.async_copy(msg, win.at[dloc], sems, add=True)

        for p in range(NPASS):
            lo = (p * NSC + cid) * WROWS
            for kk in range((TROWS + BSZ - 1) // BSZ):
                r = min(BSZ, TROWS - kk * BSZ)
                if r > 0:
                    pltpu.sync_copy(
                        zbuf.at[pl.ds(0, r)],
                        win.at[pl.ds(sid * TROWS + kk * BSZ, r)])
            for g in range(BSZ // 16):
                dloc[pl.ds(g * 16, 16)] = zi + (WROWS + sid)
            pltpu.async_copy(zbuf, win.at[dloc], sems, add=True)
            plsc.subcore_barrier()

            def chunk_body(ch, off0):
                base = sid * E_TILE + ch * CH
                pltpu.sync_copy(src_hbm.at[pl.ds(base, CH)], esrc)
                pltpu.sync_copy(dst_hbm.at[pl.ds(base, CH)], edst)

                def grp(g, off):
                    dv = edst[pl.ds(g * 16, 16)]
                    sv = esrc[pl.ds(g * 16, 16)]
                    mask = (dv >= lo) & (dv < lo + WROWS)
                    packed = sv | (dv << 16)
                    xs = jnp.where(mask, 1, 0)
                    dnums = lax.GatherDimensionNumbers(
                        offset_dims=(), collapsed_slice_dims=(0,),
                        start_index_map=(0,))
                    for kk in (1, 2, 4, 8):
                        sh = lax.gather(
                            xs, jnp.maximum(iota - kk, 0)[:, None], dnums,
                            slice_sizes=(1,),
                            mode=lax.GatherScatterMode.PROMISE_IN_BOUNDS)
                        xs = xs + jnp.where(iota >= kk, sh, 0)
                    pos = jnp.where(mask, xs + (off - 1), BSZ + 48 + iota)
                    plsc.store_scatter(pbuf, [pos], packed)
                    off = off + xs[15]

                    @pl.when(off >= BSZ)
                    def _():
                        flush(jnp.int32(BSZ), lo)
                        pk2 = pbuf[pl.ds(BSZ, 16)]
                        pbuf[pl.ds(0, 16)] = pk2
                    return jnp.where(off >= BSZ, off - BSZ, off)

                return lax.fori_loop(0, NGRP, grp, off0)

            off = lax.fori_loop(0, NCH, chunk_body, jnp.int32(0))

            @pl.when(off > 0)
            def _():
                flush(off, lo)
            pltpu.make_async_copy(msg, win.at[dloc], sems).wait()
            plsc.subcore_barrier()

            for kk in range((TROWS + BSZ - 1) // BSZ):
                r = min(BSZ, TROWS - kk * BSZ)
                if r > 0:
                    pltpu.sync_copy(win.at[pl.ds(sid * TROWS + kk * BSZ, r)],
                                    msg.at[pl.ds(0, r)])
                    pltpu.sync_copy(
                        msg.at[pl.ds(0, r)],
                        out_hbm.at[pl.ds(lo + sid * TROWS + kk * BSZ, r)])
            plsc.subcore_barrier()

    return k(P, A, src, dst)


def kernel(x, edge_index, W_e, b_e, W1, b1, W2, b2):
    P, A = _prep(x, W_e, b_e)
    aggr = _edge_sc(P, A, edge_index[0], edge_index[1])
    return _mlp(x, aggr, W1, b1, W2, b2)


# speedup vs baseline: 1.0889x; 1.0889x over previous
import functools

import jax
import jax.numpy as jnp
from jax import lax
from jax.experimental import pallas as pl
from jax.experimental.pallas import tpu as pltpu
from jax.experimental.pallas import tpu_sc as plsc

N = 50000
E = 800000
C = 5
D = 20
F_PL = 16

BN = 2000
PCOLS = 128
ACOLS = 16
GCOLS = 128
N_PAD = 50688


def _prep_body(x_ref, wi_ref, wj_ref, be_ref, p_ref, a_ref):
    xb = x_ref[...]
    wi = wi_ref[...]
    wj = wj_ref[...]
    be = be_ref[...]
    a = jnp.sum(xb * wi[None], axis=-1) + be[None]
    b = jnp.sum(xb * wj[None], axis=-1)
    rows = xb.shape[0]
    parts = [xb[:, c, :] for c in range(C)]
    parts.extend([a, b, jnp.zeros((rows, PCOLS - 100 - 2 * C), jnp.float32)])
    p_ref[...] = jnp.concatenate(parts, axis=1)
    a_ref[...] = jnp.concatenate(
        [a, jnp.zeros((rows, ACOLS - C), jnp.float32)], axis=1)


def _prep(x, W_e, b_e):
    wi = W_e[:, 0, :D]
    wj = W_e[:, 0, D:]
    be = b_e[:, 0]
    return pl.pallas_call(
        _prep_body,
        grid=(N // BN,),
        in_specs=[
            pl.BlockSpec((BN, C, D), lambda i: (i, 0, 0)),
            pl.BlockSpec((C, D), lambda i: (0, 0)),
            pl.BlockSpec((C, D), lambda i: (0, 0)),
            pl.BlockSpec((C,), lambda i: (0,)),
        ],
        out_specs=[
            pl.BlockSpec((BN, PCOLS), lambda i: (i, 0)),
            pl.BlockSpec((BN, ACOLS), lambda i: (i, 0)),
        ],
        out_shape=[
            jax.ShapeDtypeStruct((N, PCOLS), jnp.float32),
            jax.ShapeDtypeStruct((N_PAD, ACOLS), jnp.float32),
        ],
    )(x, wi, wj, be)


def _mlp_body(x_ref, g_ref, w1x_ref, w1a_ref, b1_ref, w2_ref, b2_ref, o_ref):
    xb = x_ref[...]
    gb = g_ref[...]
    for c in range(C):
        ac = gb[:, c * D:(c + 1) * D]
        h1 = jnp.tanh(
            jax.lax.dot_general(xb[:, c, :], w1x_ref[c],
                                (((1,), (1,)), ((), ())),
                                preferred_element_type=jnp.float32)
            + jax.lax.dot_general(ac, w1a_ref[c],
                                  (((1,), (1,)), ((), ())),
                                  preferred_element_type=jnp.float32)
            + b1_ref[c][None])
        h2 = jnp.tanh(
            jax.lax.dot_general(h1, w2_ref[c],
                                (((1,), (1,)), ((), ())),
                                preferred_element_type=jnp.float32)
            + b2_ref[c][None])
        o_ref[:, c, :] = h2


def _mlp(x, aggr, W1, b1, W2, b2):
    w1x = W1[:, :, :D]
    w1a = W1[:, :, D:]
    return pl.pallas_call(
        _mlp_body,
        grid=(N // BN,),
        in_specs=[
            pl.BlockSpec((BN, C, D), lambda i: (i, 0, 0)),
            pl.BlockSpec((BN, GCOLS), lambda i: (i, 0)),
            pl.BlockSpec((C, F_PL, D), lambda i: (0, 0, 0)),
            pl.BlockSpec((C, F_PL, D), lambda i: (0, 0, 0)),
            pl.BlockSpec((C, F_PL), lambda i: (0, 0)),
            pl.BlockSpec((C, F_PL, F_PL), lambda i: (0, 0, 0)),
            pl.BlockSpec((C, F_PL), lambda i: (0, 0)),
        ],
        out_specs=pl.BlockSpec((BN, C, F_PL), lambda i: (i, 0, 0)),
        out_shape=jax.ShapeDtypeStruct((N, C, F_PL), jnp.float32),
    )(x, aggr, w1x, w1a, b1, W2, b2)


WROWS = 8448
NPASS = 3
NSC = 2
BSZ = 96
CH = 2000
E_TILE = E // 16
NCH = E_TILE // CH
NGRP = CH // 16
TROWS = WROWS // 16


def _edge_sc(P, A, src, dst):
    mesh = plsc.VectorSubcoreMesh(core_axis_name="c", subcore_axis_name="s")

    @functools.partial(
        pl.kernel, mesh=mesh,
        out_type=jax.ShapeDtypeStruct((N_PAD, GCOLS), jnp.float32),
        scratch_types=[
            pltpu.VMEM((CH,), jnp.int32),
            pltpu.VMEM((CH,), jnp.int32),
            pltpu.VMEM((BSZ + 64,), jnp.int32),
            pltpu.VMEM((BSZ,), jnp.int32),
            pltpu.VMEM((BSZ,), jnp.int32),
            pltpu.VMEM((BSZ,), jnp.int32),
            pltpu.VMEM((BSZ, PCOLS), jnp.float32),
            pltpu.VMEM((BSZ, PCOLS), jnp.float32),
            pltpu.VMEM((BSZ, GCOLS), jnp.float32),
            pltpu.VMEM((BSZ, GCOLS), jnp.float32),
            pltpu.VMEM_SHARED((WROWS + 16, GCOLS), jnp.float32),
            pltpu.SemaphoreType.DMA,
            pltpu.SemaphoreType.DMA,
        ],
        compiler_params=pltpu.CompilerParams(needs_layout_passes=False),
    )
    def k(p_hbm, a_hbm, src_hbm, dst_hbm, out_hbm,
          esrc, edst, pbuf, sidx, didx, dloc, prow, drow, msg, zbuf,
          win, sem, sems):
        cid = lax.axis_index("c")
        sid = lax.axis_index("s")
        iota = lax.iota(jnp.int32, 16)
        zi = jnp.zeros((16,), jnp.int32)
        zf = jnp.zeros((16,), jnp.float32)

        for g in range((BSZ + 64) // 16):
            pbuf[pl.ds(g * 16, 16)] = zi

        def _zrow(r, _):
            for kk in range(GCOLS // 16):
                zbuf[r, pl.ds(kk * 16, 16)] = zf
            return 0
        lax.fori_loop(0, BSZ, _zrow, 0)

        def flush(nact, lo):
            pltpu.make_async_copy(msg, win.at[dloc], sems).wait()
            for g in range(BSZ // 16):
                pk = pbuf[pl.ds(g * 16, 16)]
                s = pk & 0xFFFF
                d = (pk >> 16) & 0xFFFF
                act = (iota + (g * 16)) < nact
                sidx[pl.ds(g * 16, 16)] = s
                didx[pl.ds(g * 16, 16)] = d
                dloc[pl.ds(g * 16, 16)] = jnp.where(act, d - lo, WROWS + sid)
            d1 = pltpu.async_copy(p_hbm.at[didx], drow, sem)
            d2 = pltpu.async_copy(p_hbm.at[sidx], prow, sem)
            d1.wait()
            d2.wait()
            for g in range(BSZ // 16):
                rid = iota + (g * 16)
                ev = [plsc.load_gather(drow, [rid, zi + (100 + c)])
                      + plsc.load_gather(prow, [rid, zi + (105 + c)])
                      for c in range(C)]
                m = ev[0]
                for c in range(1, C):
                    m = jnp.maximum(m, ev[c])
                ex = [jnp.exp(e - m) for e in ev]
                s = ex[0]
                for c in range(1, C):
                    s = s + ex[c]
                w = [e / s for e in ex]

                @plsc.parallel_loop(0, D, unroll=4)
                def _(f):
                    for c in range(C):
                        colv = zi + (c * D) + f
                        xv = plsc.load_gather(prow, [rid, colv])
                        plsc.store_scatter(msg, [rid, colv], xv * w[c])
            pltpu.async_copy(msg, win.at[dloc], sems, add=True)

        for p in range(NPASS):
            lo = (p * NSC + cid) * WROWS
            for kk in range((TROWS + BSZ - 1) // BSZ):
                r = min(BSZ, TROWS - kk * BSZ)
                if r > 0:
                    pltpu.sync_copy(
                        zbuf.at[pl.ds(0, r)],
                        win.at[pl.ds(sid * TROWS + kk * BSZ, r)])
            for g in range(BSZ // 16):
                dloc[pl.ds(g * 16, 16)] = zi + (WROWS + sid)
            pltpu.async_copy(zbuf, win.at[dloc], sems, add=True)
            plsc.subcore_barrier()

            def chunk_body(ch, off0):
                base = sid * E_TILE + ch * CH
                pltpu.sync_copy(src_hbm.at[pl.ds(base, CH)], esrc)
                pltpu.sync_copy(dst_hbm.at[pl.ds(base, CH)], edst)

                def grp(g, off):
                    dv = edst[pl.ds(g * 16, 16)]
                    sv = esrc[pl.ds(g * 16, 16)]
                    mask = (dv >= lo) & (dv < lo + WROWS)
                    packed = sv | (dv << 16)
                    xs = jnp.where(mask, 1, 0)
                    dnums = lax.GatherDimensionNumbers(
                        offset_dims=(), collapsed_slice_dims=(0,),
                        start_index_map=(0,))
                    for kk in (1, 2, 4, 8):
                        sh = lax.gather(
                            xs, jnp.maximum(iota - kk, 0)[:, None], dnums,
                            slice_sizes=(1,),
                            mode=lax.GatherScatterMode.PROMISE_IN_BOUNDS)
                        xs = xs + jnp.where(iota >= kk, sh, 0)
                    pos = jnp.where(mask, xs + (off - 1), BSZ + 48 + iota)
                    plsc.store_scatter(pbuf, [pos], packed)
                    off = off + xs[15]

                    @pl.when(off >= BSZ)
                    def _():
                        flush(jnp.int32(BSZ), lo)
                        pk2 = pbuf[pl.ds(BSZ, 16)]
                        pbuf[pl.ds(0, 16)] = pk2
                    return jnp.where(off >= BSZ, off - BSZ, off)

                return lax.fori_loop(0, NGRP, grp, off0)

            off = lax.fori_loop(0, NCH, chunk_body, jnp.int32(0))

            @pl.when(off > 0)
            def _():
                flush(off, lo)
            pltpu.make_async_copy(msg, win.at[dloc], sems).wait()
            plsc.subcore_barrier()

            for kk in range((TROWS + BSZ - 1) // BSZ):
                r = min(BSZ, TROWS - kk * BSZ)
                if r > 0:
                    pltpu.sync_copy(win.at[pl.ds(sid * TROWS + kk * BSZ, r)],
                                    msg.at[pl.ds(0, r)])
                    pltpu.sync_copy(
                        msg.at[pl.ds(0, r)],
                        out_hbm.at[pl.ds(lo + sid * TROWS + kk * BSZ, r)])
            plsc.subcore_barrier()

    return k(P, A, src, dst)


def kernel(x, edge_index, W_e, b_e, W1, b1, W2, b2):
    P, A = _prep(x, W_e, b_e)
    aggr = _edge_sc(P, A, edge_index[0], edge_index[1])
    return _mlp(x, aggr, W1, b1, W2, b2)
